# baseline (speedup 1.0000x reference)
import functools

import jax
import jax.numpy as jnp
from jax import lax
from jax.experimental import pallas as pl
from jax.experimental.pallas import tpu as pltpu
from jax.experimental.pallas import tpu_sc as plsc

N = 10000
E = 320000
D = 128
LANES = 16
NC = 2
NS = 16
NW = NC * NS

C = 64
NCHUNKS = E // C
NCW0 = NCHUNKS // NW
NCREM = NCHUNKS % NW
NPAD = 10240
ROWS_PT = NPAD // NS
ZROWS = 128


def _sc_propagate(x, meta, rel):
    mesh = plsc.VectorSubcoreMesh(core_axis_name="c", subcore_axis_name="s")

    @functools.partial(
        pl.kernel,
        out_type=jax.ShapeDtypeStruct((NC, NPAD, D), jnp.float32),
        mesh=mesh,
        scratch_types=dict(
            h_sh=pltpu.VMEM_SHARED((NPAD, D), jnp.float32),
            metab=pltpu.VMEM((3, 3, C), jnp.int32),
            xrb=pltpu.VMEM((3 * C, D), jnp.float32),
            relb=pltpu.VMEM((2 * C, D), jnp.float32),
            sem_meta=pltpu.SemaphoreType.DMA((3,)),
            sem_rel=pltpu.SemaphoreType.DMA((2,)),
            sem_g=pltpu.SemaphoreType.DMA((3,)),
            sem_sc=pltpu.SemaphoreType.DMA((3,)),
        ),
    )
    def k(x_hbm, meta_hbm, rel_hbm, out_hbm,
          h_sh, metab, xrb, relb, sem_meta, sem_rel, sem_g, sem_sc):
        cid = lax.axis_index("c")
        sid = lax.axis_index("s")
        wid = sid * NC + cid
        ncw = NCW0 + (wid < NCREM).astype(jnp.int32)
        cstart = NCW0 * wid + jnp.minimum(wid, NCREM)

        def zrow(r, _):
            for kk in range(D // LANES):
                xrb[r, pl.ds(kk * LANES, LANES)] = jnp.zeros(
                    (LANES,), jnp.float32)
            return 0
        lax.fori_loop(0, ZROWS, zrow, 0)
        for j in range(ROWS_PT // ZROWS):
            pltpu.sync_copy(xrb.at[pl.ds(0, ZROWS)],
                            h_sh.at[pl.ds(sid * ROWS_PT + j * ZROWS,
                                          ZROWS)])
        plsc.subcore_barrier()

        def start_meta(k_, b3):
            pltpu.async_copy(meta_hbm.at[cstart + k_], metab.at[b3],
                             sem_meta.at[b3])

        def wait_meta(k_, b3):
            pltpu.make_async_copy(meta_hbm.at[cstart + k_], metab.at[b3],
                                  sem_meta.at[b3]).wait()

        def start_rel(k_, b2):
            pltpu.async_copy(rel_hbm.at[pl.ds((cstart + k_) * C, C)],
                             relb.at[pl.ds(b2 * C, C)], sem_rel.at[b2])

        def wait_rel(k_, b2):
            pltpu.make_async_copy(rel_hbm.at[pl.ds((cstart + k_) * C, C)],
                                  relb.at[pl.ds(b2 * C, C)],
                                  sem_rel.at[b2]).wait()

        def start_gather(b3):
            pltpu.async_copy(x_hbm.at[metab.at[b3, 0]],
                             xrb.at[pl.ds(b3 * C, C)], sem_g.at[b3])

        def wait_gather(b3):
            pltpu.make_async_copy(x_hbm.at[metab.at[b3, 0]],
                                  xrb.at[pl.ds(b3 * C, C)],
                                  sem_g.at[b3]).wait()

        def start_scatter(b3):
            pltpu.async_copy(xrb.at[pl.ds(b3 * C, C)],
                             h_sh.at[metab.at[b3, 1]], sem_sc.at[b3],
                             add=True)

        def wait_scatter(b3):
            pltpu.make_async_copy(xrb.at[pl.ds(b3 * C, C)],
                                  h_sh.at[metab.at[b3, 1]],
                                  sem_sc.at[b3]).wait()

        start_meta(0, 0)
        start_rel(0, 0)
        start_meta(1, 1)
        start_rel(1, 1)
        wait_meta(0, 0)
        wait_rel(0, 0)
        start_gather(0)

        def chunk(k_, carry):
            p, q, r, b2 = carry
            b2n = 1 - b2
            wait_gather(p)

            @pl.when(k_ + 1 < ncw)
            def _():
                wait_meta(k_ + 1, q)
                wait_rel(k_ + 1, b2n)
                start_gather(q)

            bo = p * C
            ro = b2 * C

            @plsc.parallel_loop(0, C // LANES)
            def group(g):
                nvec = lax.bitcast_convert_type(
                    metab[p, 2, pl.ds(g * LANES, LANES)], jnp.float32)
                for j in range(LANES):
                    nv = nvec[j]
                    e = bo + g * LANES + j
                    er = ro + g * LANES + j
                    for kk in range(D // LANES):
                        sl = pl.ds(kk * LANES, LANES)
                        xrb[e, sl] = relb[er, sl] * xrb[e, sl] * nv

            start_scatter(p)

            @pl.when(k_ + 2 < ncw)
            def _():
                @pl.when(k_ >= 1)
                def _():
                    wait_scatter(r)
                start_meta(k_ + 2, r)
                start_rel(k_ + 2, b2)
            return (q, r, p, b2n)
        lax.fori_loop(0, ncw, chunk,
                      (jnp.int32(0), jnp.int32(1), jnp.int32(2),
                       jnp.int32(0)))

        wait_scatter(0)
        wait_scatter(1)
        wait_scatter(2)
        plsc.subcore_barrier()

        for j in range(ROWS_PT // ZROWS):
            r0 = sid * ROWS_PT + j * ZROWS
            pltpu.sync_copy(h_sh.at[pl.ds(r0, ZROWS)],
                            out_hbm.at[cid, pl.ds(r0, ZROWS)])

    return k(x, meta, rel)


def _tc_finish_body(parts_ref, tgt_ref, w_ref, b_ref, out_ref):
    h = parts_ref[0] + parts_ref[1] + tgt_ref[...]
    y = lax.dot_general(h, w_ref[...], (((1,), (1,)), ((), ())),
                        preferred_element_type=jnp.float32)
    out_ref[...] = jnp.maximum(y + b_ref[...], 0.0)


def _tc_finish(parts, target, W, b2):
    BR = 1000
    grid = (N // BR,)
    return pl.pallas_call(
        _tc_finish_body,
        grid=grid,
        in_specs=[
            pl.BlockSpec((NC, BR, D), lambda i: (0, i, 0)),
            pl.BlockSpec((BR, D), lambda i: (i, 0)),
            pl.BlockSpec((D, D), lambda i: (0, 0)),
            pl.BlockSpec((1, D), lambda i: (0, 0)),
        ],
        out_specs=pl.BlockSpec((BR, D), lambda i: (i, 0)),
        out_shape=jax.ShapeDtypeStruct((N, D), jnp.float32),
    )(parts, target, W, b2)


def kernel(x, edge_index, norm, edge_rel_emd, target_rel_emd_new, W_line,
           b_line):
    src = edge_index[0].astype(jnp.int32)
    dst = edge_index[1].astype(jnp.int32)
    norm_bits = lax.bitcast_convert_type(norm.reshape(E), jnp.int32)
    meta = (jnp.stack([src, dst, norm_bits], axis=0)
            .reshape(3, E // C, C).transpose(1, 0, 2))
    parts = _sc_propagate(x, meta, edge_rel_emd)
    return _tc_finish(parts, target_rel_emd_new, W_line,
                      b_line.reshape(1, D))


# speedup vs baseline: 1.7412x; 1.7412x over previous
import functools

import jax
import jax.numpy as jnp
from jax import lax
from jax.experimental import pallas as pl
from jax.experimental.pallas import tpu as pltpu
from jax.experimental.pallas import tpu_sc as plsc

N = 10000
E = 320000
D = 128
LANES = 16
NC = 2
NS = 16
NW = NC * NS

C = 80
EPW = E // NW
NCHUNK = EPW // C
NBUF = 2
NPAD = 10240
ROWS_PT = NPAD // NS
ZROWS = 128


def _sc_propagate(x, meta, rel):
    mesh = plsc.VectorSubcoreMesh(core_axis_name="c", subcore_axis_name="s")

    @functools.partial(
        pl.kernel,
        out_type=jax.ShapeDtypeStruct((NC, NPAD, D), jnp.float32),
        mesh=mesh,
        scratch_types=dict(
            h_sh=pltpu.VMEM_SHARED((NPAD, D), jnp.float32),
            metab=pltpu.VMEM((NBUF, 3, C), jnp.int32),
            xrb=pltpu.VMEM((NBUF * C, D), jnp.float32),
            relb=pltpu.VMEM((NBUF * C, D), jnp.float32),
            sem_lin=pltpu.SemaphoreType.DMA((NBUF,)),
            sem_g=pltpu.SemaphoreType.DMA((NBUF,)),
            sem_sc=pltpu.SemaphoreType.DMA((NBUF,)),
        ),
    )
    def k(x_hbm, meta_hbm, rel_hbm, out_hbm,
          h_sh, metab, xrb, relb,
          sem_lin, sem_g, sem_sc):
        cid = lax.axis_index("c")
        sid = lax.axis_index("s")
        wid = sid * NC + cid

        def zrow(r, _):
            for kk in range(D // LANES):
                xrb[r, pl.ds(kk * LANES, LANES)] = jnp.zeros(
                    (LANES,), jnp.float32)
            return 0
        lax.fori_loop(0, ZROWS, zrow, 0)
        for j in range(ROWS_PT // ZROWS):
            pltpu.sync_copy(xrb.at[pl.ds(0, ZROWS)],
                            h_sh.at[pl.ds(sid * ROWS_PT + j * ZROWS,
                                          ZROWS)])
        plsc.subcore_barrier()

        def start_lin(ci, b):
            base = wid * EPW + ci * C
            pltpu.async_copy(meta_hbm.at[wid * NCHUNK + ci], metab.at[b],
                             sem_lin.at[b])
            pltpu.async_copy(rel_hbm.at[pl.ds(base, C)],
                             relb.at[pl.ds(b * C, C)], sem_lin.at[b])

        def wait_lin(ci, b):
            base = wid * EPW + ci * C
            pltpu.make_async_copy(meta_hbm.at[wid * NCHUNK + ci], metab.at[b],
                                  sem_lin.at[b]).wait()
            pltpu.make_async_copy(rel_hbm.at[pl.ds(base, C)],
                                  relb.at[pl.ds(b * C, C)],
                                  sem_lin.at[b]).wait()

        def start_gather(b):
            pltpu.async_copy(x_hbm.at[metab.at[b, 0]], xrb.at[pl.ds(b * C, C)],
                             sem_g.at[b])

        def wait_gather(b):
            pltpu.make_async_copy(x_hbm.at[metab.at[b, 0]],
                                  xrb.at[pl.ds(b * C, C)],
                                  sem_g.at[b]).wait()

        def start_scatter(b):
            pltpu.async_copy(relb.at[pl.ds(b * C, C)],
                             h_sh.at[metab.at[b, 1]], sem_sc.at[b], add=True)

        def wait_scatter(b):
            pltpu.make_async_copy(relb.at[pl.ds(b * C, C)],
                                  h_sh.at[metab.at[b, 1]], sem_sc.at[b]).wait()

        start_lin(0, 0)
        start_lin(1, 1)
        wait_lin(0, 0)
        start_gather(0)

        def chunk(ci, _):
            b = lax.rem(ci, NBUF)
            wait_gather(b)

            @pl.when(ci + 1 < NCHUNK)
            def _():
                b1 = lax.rem(ci + 1, NBUF)
                wait_lin(ci + 1, b1)
                start_gather(b1)

            bo = b * C

            @plsc.parallel_loop(0, C // LANES)
            def group(g):
                nvec = lax.bitcast_convert_type(
                    metab[b, 2, pl.ds(g * LANES, LANES)], jnp.float32)
                for j in range(LANES):
                    nv = nvec[j]
                    e = bo + g * LANES + j
                    for kk in range(D // LANES):
                        sl = pl.ds(kk * LANES, LANES)
                        relb[e, sl] = relb[e, sl] * xrb[e, sl] * nv

            start_scatter(b)

            @pl.when(ci + 2 < NCHUNK)
            def _():
                wait_scatter(b)
                start_lin(ci + 2, b)
            return 0
        lax.fori_loop(0, NCHUNK, chunk, 0)

        wait_scatter(lax.rem(NCHUNK - 2, NBUF))
        wait_scatter(lax.rem(NCHUNK - 1, NBUF))
        plsc.subcore_barrier()

        for j in range(ROWS_PT // ZROWS):
            r0 = sid * ROWS_PT + j * ZROWS
            pltpu.sync_copy(h_sh.at[pl.ds(r0, ZROWS)],
                            out_hbm.at[cid, pl.ds(r0, ZROWS)])

    return k(x, meta, rel)


def _tc_finish_body(parts_ref, tgt_ref, w_ref, b_ref, out_ref):
    h = parts_ref[0] + parts_ref[1] + tgt_ref[...]
    y = lax.dot_general(h, w_ref[...], (((1,), (1,)), ((), ())),
                        preferred_element_type=jnp.float32)
    out_ref[...] = jnp.maximum(y + b_ref[...], 0.0)


def _tc_finish(parts, target, W, b2):
    BR = 1000
    grid = (N // BR,)
    return pl.pallas_call(
        _tc_finish_body,
        grid=grid,
        in_specs=[
            pl.BlockSpec((NC, BR, D), lambda i: (0, i, 0)),
            pl.BlockSpec((BR, D), lambda i: (i, 0)),
            pl.BlockSpec((D, D), lambda i: (0, 0)),
            pl.BlockSpec((1, D), lambda i: (0, 0)),
        ],
        out_specs=pl.BlockSpec((BR, D), lambda i: (i, 0)),
        out_shape=jax.ShapeDtypeStruct((N, D), jnp.float32),
    )(parts, target, W, b2)


def kernel(x, edge_index, norm, edge_rel_emd, target_rel_emd_new, W_line,
           b_line):
    src = edge_index[0].astype(jnp.int32)
    dst = edge_index[1].astype(jnp.int32)
    norm_bits = lax.bitcast_convert_type(norm.reshape(E), jnp.int32)
    meta = (jnp.stack([src, dst, norm_bits], axis=0)
            .reshape(3, E // C, C).transpose(1, 0, 2))
    parts = _sc_propagate(x, meta, edge_rel_emd)
    return _tc_finish(parts, target_rel_emd_new, W_line,
                      b_line.reshape(1, D))


# speedup vs baseline: 1.7432x; 1.0011x over previous
import functools

import jax
import jax.numpy as jnp
from jax import lax
from jax.experimental import pallas as pl
from jax.experimental.pallas import tpu as pltpu
from jax.experimental.pallas import tpu_sc as plsc

N = 10000
E = 320000
D = 128
LANES = 16
NC = 2
NS = 16
NW = NC * NS

C = 80
EPW = E // NW
NCHUNK = EPW // C
NBUF = 2
NPAD = 10240
ROWS_PT = NPAD // NS
ZROWS = 128


def _sc_propagate(x, meta, rel):
    mesh = plsc.VectorSubcoreMesh(core_axis_name="c", subcore_axis_name="s")

    @functools.partial(
        pl.kernel,
        out_type=jax.ShapeDtypeStruct((NC, NPAD, D), jnp.float32),
        mesh=mesh,
        scratch_types=dict(
            h_sh=pltpu.VMEM_SHARED((NPAD, D), jnp.float32),
            metab=pltpu.VMEM((NBUF, 3, C), jnp.int32),
            xrb=pltpu.VMEM((NBUF * C, D), jnp.float32),
            relb=pltpu.VMEM((NBUF * C, D), jnp.float32),
            sem_lin=pltpu.SemaphoreType.DMA((NBUF,)),
            sem_g=pltpu.SemaphoreType.DMA((NBUF,)),
            sem_sc=pltpu.SemaphoreType.DMA((NBUF,)),
        ),
    )
    def k(x_hbm, meta_hbm, rel_hbm, out_hbm,
          h_sh, metab, xrb, relb,
          sem_lin, sem_g, sem_sc):
        cid = lax.axis_index("c")
        sid = lax.axis_index("s")
        wid = sid * NC + cid

        def zrow(r, _):
            for kk in range(D // LANES):
                xrb[r, pl.ds(kk * LANES, LANES)] = jnp.zeros(
                    (LANES,), jnp.float32)
            return 0
        lax.fori_loop(0, ZROWS, zrow, 0)
        for j in range(ROWS_PT // ZROWS):
            pltpu.sync_copy(xrb.at[pl.ds(0, ZROWS)],
                            h_sh.at[pl.ds(sid * ROWS_PT + j * ZROWS,
                                          ZROWS)])
        plsc.subcore_barrier()

        def start_lin(ci, b):
            base = wid * EPW + ci * C
            pltpu.async_copy(meta_hbm.at[wid * NCHUNK + ci], metab.at[b],
                             sem_lin.at[b])
            pltpu.async_copy(rel_hbm.at[pl.ds(base, C)],
                             relb.at[pl.ds(b * C, C)], sem_lin.at[b])

        def wait_lin(ci, b):
            base = wid * EPW + ci * C
            pltpu.make_async_copy(meta_hbm.at[wid * NCHUNK + ci], metab.at[b],
                                  sem_lin.at[b]).wait()
            pltpu.make_async_copy(rel_hbm.at[pl.ds(base, C)],
                                  relb.at[pl.ds(b * C, C)],
                                  sem_lin.at[b]).wait()

        def start_gather(b):
            pltpu.async_copy(x_hbm.at[metab.at[b, 0]], xrb.at[pl.ds(b * C, C)],
                             sem_g.at[b])

        def wait_gather(b):
            pltpu.make_async_copy(x_hbm.at[metab.at[b, 0]],
                                  xrb.at[pl.ds(b * C, C)],
                                  sem_g.at[b]).wait()

        def start_scatter(b):
            pltpu.async_copy(relb.at[pl.ds(b * C, C)],
                             h_sh.at[metab.at[b, 1]], sem_sc.at[b], add=True)

        def wait_scatter(b):
            pltpu.make_async_copy(relb.at[pl.ds(b * C, C)],
                                  h_sh.at[metab.at[b, 1]], sem_sc.at[b]).wait()

        start_lin(0, 0)
        start_lin(1, 1)
        wait_lin(0, 0)
        start_gather(0)

        def chunk(ci, _):
            b = lax.rem(ci, NBUF)
            wait_gather(b)

            @pl.when(ci + 1 < NCHUNK)
            def _():
                b1 = lax.rem(ci + 1, NBUF)
                wait_lin(ci + 1, b1)
                start_gather(b1)

            bo = b * C

            @plsc.parallel_loop(0, C // LANES)
            def group(g):
                nvec = lax.bitcast_convert_type(
                    metab[b, 2, pl.ds(g * LANES, LANES)], jnp.float32)
                for j in range(LANES):
                    nv = nvec[j]
                    e = bo + g * LANES + j
                    for kk in range(D // LANES):
                        sl = pl.ds(kk * LANES, LANES)
                        relb[e, sl] = relb[e, sl] * xrb[e, sl] * nv

            pltpu.touch(relb)
            start_scatter(b)

            @pl.when(ci + 2 < NCHUNK)
            def _():
                wait_scatter(b)
                start_lin(ci + 2, b)
            return 0
        lax.fori_loop(0, NCHUNK, chunk, 0)

        wait_scatter(lax.rem(NCHUNK - 2, NBUF))
        wait_scatter(lax.rem(NCHUNK - 1, NBUF))
        plsc.subcore_barrier()

        for j in range(ROWS_PT // ZROWS):
            r0 = sid * ROWS_PT + j * ZROWS
            pltpu.sync_copy(h_sh.at[pl.ds(r0, ZROWS)],
                            out_hbm.at[cid, pl.ds(r0, ZROWS)])

    return k(x, meta, rel)


def _tc_finish_body(parts_ref, tgt_ref, w_ref, b_ref, out_ref):
    h = parts_ref[0] + parts_ref[1] + tgt_ref[...]
    y = lax.dot_general(h, w_ref[...], (((1,), (1,)), ((), ())),
                        preferred_element_type=jnp.float32)
    out_ref[...] = jnp.maximum(y + b_ref[...], 0.0)


def _tc_finish(parts, target, W, b2):
    BR = 1000
    grid = (N // BR,)
    return pl.pallas_call(
        _tc_finish_body,
        grid=grid,
        in_specs=[
            pl.BlockSpec((NC, BR, D), lambda i: (0, i, 0)),
            pl.BlockSpec((BR, D), lambda i: (i, 0)),
            pl.BlockSpec((D, D), lambda i: (0, 0)),
            pl.BlockSpec((1, D), lambda i: (0, 0)),
        ],
        out_specs=pl.BlockSpec((BR, D), lambda i: (i, 0)),
        out_shape=jax.ShapeDtypeStruct((N, D), jnp.float32),
    )(parts, target, W, b2)


def kernel(x, edge_index, norm, edge_rel_emd, target_rel_emd_new, W_line,
           b_line):
    src = edge_index[0].astype(jnp.int32)
    dst = edge_index[1].astype(jnp.int32)
    norm_bits = lax.bitcast_convert_type(norm.reshape(E), jnp.int32)
    meta = (jnp.stack([src, dst, norm_bits], axis=0)
            .reshape(3, E // C, C).transpose(1, 0, 2))
    parts = _sc_propagate(x, meta, edge_rel_emd)
    return _tc_finish(parts, target_rel_emd_new, W_line,
                      b_line.reshape(1, D))


# speedup vs baseline: 1.9959x; 1.1450x over previous
import functools

import jax
import jax.numpy as jnp
from jax import lax
from jax.experimental import pallas as pl
from jax.experimental.pallas import tpu as pltpu
from jax.experimental.pallas import tpu_sc as plsc

N = 10000
E = 320000
D = 128
LANES = 16
NC = 2
NS = 16
NW = NC * NS

C = 80
EPW = E // NW
NCHUNK = EPW // C
NBUF = 2
NPAD = 10240
ROWS_PT = NPAD // NS
ZROWS = 128


def _sc_propagate(x, meta, rel):
    mesh = plsc.VectorSubcoreMesh(core_axis_name="c", subcore_axis_name="s")

    @functools.partial(
        pl.kernel,
        out_type=jax.ShapeDtypeStruct((NC, NPAD, D), jnp.float32),
        mesh=mesh,
        scratch_types=dict(
            h_sh=pltpu.VMEM_SHARED((NPAD, D), jnp.float32),
            metab=pltpu.VMEM((NBUF, 3, C), jnp.int32),
            xrb=pltpu.VMEM((NBUF * C, D), jnp.float32),
            relb=pltpu.VMEM((NBUF * C, D), jnp.float32),
            sem_lin=pltpu.SemaphoreType.DMA((NBUF,)),
            sem_g=pltpu.SemaphoreType.DMA((NBUF,)),
            sem_sc=pltpu.SemaphoreType.DMA((NBUF,)),
        ),
    )
    def k(x_hbm, meta_hbm, rel_hbm, out_hbm,
          h_sh, metab, xrb, relb,
          sem_lin, sem_g, sem_sc):
        cid = lax.axis_index("c")
        sid = lax.axis_index("s")
        wid = sid * NC + cid

        def zrow(r, _):
            for kk in range(D // LANES):
                xrb[r, pl.ds(kk * LANES, LANES)] = jnp.zeros(
                    (LANES,), jnp.float32)
            return 0
        lax.fori_loop(0, ZROWS, zrow, 0)
        for j in range(ROWS_PT // ZROWS):
            pltpu.sync_copy(xrb.at[pl.ds(0, ZROWS)],
                            h_sh.at[pl.ds(sid * ROWS_PT + j * ZROWS,
                                          ZROWS)])
        plsc.subcore_barrier()

        def start_lin(ci, b):
            base = wid * EPW + ci * C
            pltpu.async_copy(meta_hbm.at[wid * NCHUNK + ci], metab.at[b],
                             sem_lin.at[b])
            pltpu.async_copy(rel_hbm.at[pl.ds(base, C)],
                             relb.at[pl.ds(b * C, C)], sem_lin.at[b])

        def wait_lin(ci, b):
            base = wid * EPW + ci * C
            pltpu.make_async_copy(meta_hbm.at[wid * NCHUNK + ci], metab.at[b],
                                  sem_lin.at[b]).wait()
            pltpu.make_async_copy(rel_hbm.at[pl.ds(base, C)],
                                  relb.at[pl.ds(b * C, C)],
                                  sem_lin.at[b]).wait()

        def start_gather(b):
            pltpu.async_copy(x_hbm.at[metab.at[b, 0]], xrb.at[pl.ds(b * C, C)],
                             sem_g.at[b])

        def wait_gather(b):
            pltpu.make_async_copy(x_hbm.at[metab.at[b, 0]],
                                  xrb.at[pl.ds(b * C, C)],
                                  sem_g.at[b]).wait()

        def start_scatter(b):
            pltpu.async_copy(xrb.at[pl.ds(b * C, C)],
                             h_sh.at[metab.at[b, 1]], sem_sc.at[b], add=True)

        def wait_scatter(b):
            pltpu.make_async_copy(xrb.at[pl.ds(b * C, C)],
                                  h_sh.at[metab.at[b, 1]], sem_sc.at[b]).wait()

        start_lin(0, 0)
        start_lin(1, 1)
        wait_lin(0, 0)
        start_gather(0)

        def chunk(ci, _):
            b = lax.rem(ci, NBUF)
            wait_gather(b)

            @pl.when(ci + 1 < NCHUNK)
            def _():
                b1 = lax.rem(ci + 1, NBUF)
                wait_lin(ci + 1, b1)

                @pl.when(ci >= 1)
                def _():
                    wait_scatter(b1)
                start_gather(b1)

            bo = b * C

            @plsc.parallel_loop(0, C // LANES)
            def group(g):
                nvec = lax.bitcast_convert_type(
                    metab[b, 2, pl.ds(g * LANES, LANES)], jnp.float32)
                for j in range(LANES):
                    nv = nvec[j]
                    e = bo + g * LANES + j
                    for kk in range(D // LANES):
                        sl = pl.ds(kk * LANES, LANES)
                        xrb[e, sl] = relb[e, sl] * xrb[e, sl] * nv

            pltpu.touch(xrb)
            start_scatter(b)

            @pl.when(ci + 2 < NCHUNK)
            def _():
                start_lin(ci + 2, b)
            return 0
        lax.fori_loop(0, NCHUNK, chunk, 0)

        wait_scatter(lax.rem(NCHUNK - 2, NBUF))
        wait_scatter(lax.rem(NCHUNK - 1, NBUF))
        plsc.subcore_barrier()

        for j in range(ROWS_PT // ZROWS):
            r0 = sid * ROWS_PT + j * ZROWS
            pltpu.sync_copy(h_sh.at[pl.ds(r0, ZROWS)],
                            out_hbm.at[cid, pl.ds(r0, ZROWS)])

    return k(x, meta, rel)


def _tc_finish_body(parts_ref, tgt_ref, w_ref, b_ref, out_ref):
    h = parts_ref[0] + parts_ref[1] + tgt_ref[...]
    y = lax.dot_general(h, w_ref[...], (((1,), (1,)), ((), ())),
                        preferred_element_type=jnp.float32)
    out_ref[...] = jnp.maximum(y + b_ref[...], 0.0)


def _tc_finish(parts, target, W, b2):
    BR = 1000
    grid = (N // BR,)
    return pl.pallas_call(
        _tc_finish_body,
        grid=grid,
        in_specs=[
            pl.BlockSpec((NC, BR, D), lambda i: (0, i, 0)),
            pl.BlockSpec((BR, D), lambda i: (i, 0)),
            pl.BlockSpec((D, D), lambda i: (0, 0)),
            pl.BlockSpec((1, D), lambda i: (0, 0)),
        ],
        out_specs=pl.BlockSpec((BR, D), lambda i: (i, 0)),
        out_shape=jax.ShapeDtypeStruct((N, D), jnp.float32),
    )(parts, target, W, b2)


def kernel(x, edge_index, norm, edge_rel_emd, target_rel_emd_new, W_line,
           b_line):
    src = edge_index[0].astype(jnp.int32)
    dst = edge_index[1].astype(jnp.int32)
    norm_bits = lax.bitcast_convert_type(norm.reshape(E), jnp.int32)
    meta = (jnp.stack([src, dst, norm_bits], axis=0)
            .reshape(3, E // C, C).transpose(1, 0, 2))
    parts = _sc_propagate(x, meta, edge_rel_emd)
    return _tc_finish(parts, target_rel_emd_new, W_line,
                      b_line.reshape(1, D))
